# baseline (speedup 1.0000x reference)
import functools

import jax
import jax.numpy as jnp
from jax import lax
from jax.experimental import pallas as pl
from jax.experimental.pallas import tpu as pltpu
from jax.experimental.pallas import tpu_sc as plsc

_NC = 2
_NS = 16
_NW = _NC * _NS
_CH = 128

_CLIP_LO = 1e-8
_CLIP_HI = 1.0 - 1e-8
_LANES = 128


def _sc_gather_body(src_hbm, idx_hbm, out_hbm, idx_v, vals_v, sem):
    wid = lax.axis_index("s") * _NC + lax.axis_index("c")
    npw = idx_v.shape[0]
    span = pl.ds(wid * npw, npw)
    pltpu.sync_copy(idx_hbm.at[span], idx_v)
    nch = npw // _CH

    def fire(c, carry):
        sl = pl.ds(c * _CH, _CH)
        pltpu.async_copy(src_hbm.at[idx_v.at[sl]], vals_v.at[sl], sem)
        return carry

    def drain(c, carry):
        sl = pl.ds(c * _CH, _CH)
        pltpu.make_async_copy(src_hbm.at[idx_v.at[sl]], vals_v.at[sl], sem).wait()
        return carry

    lax.fori_loop(0, nch, fire, 0)
    lax.fori_loop(0, nch, drain, 0)
    pltpu.sync_copy(vals_v, out_hbm.at[span])


def _make_sc_gather(n_elems):
    npw = n_elems // _NW
    assert npw % _CH == 0
    assert (npw * _NW) == n_elems and npw % 8 == 0
    return functools.partial(
        pl.kernel,
        out_type=jax.ShapeDtypeStruct((n_elems,), jnp.float32),
        mesh=plsc.VectorSubcoreMesh(core_axis_name="c", subcore_axis_name="s"),
        scratch_types=[
            pltpu.VMEM((npw,), jnp.int32),
            pltpu.VMEM((npw,), jnp.float32),
            pltpu.SemaphoreType.DMA,
        ],
    )(_sc_gather_body)


def _tc_dense_body(src_ref, sum_ref, acc_ref):
    i = pl.program_id(0)
    n_i = pl.num_programs(0)

    p = jnp.clip(src_ref[...], _CLIP_LO, _CLIP_HI)
    dense = jnp.sum(jnp.log(1.0 - p))

    @pl.when(i == 0)
    def _():
        acc_ref[0] = 0.0

    acc_ref[0] += -dense

    @pl.when(i == n_i - 1)
    def _():
        sum_ref[0, 0] = acc_ref[0]


def _tc_dense(src_t, block_rows=200):
    v, b = src_t.shape
    grid = (v // block_rows,)
    return pl.pallas_call(
        _tc_dense_body,
        grid=grid,
        in_specs=[pl.BlockSpec((block_rows, b), lambda i: (i, 0))],
        out_specs=pl.BlockSpec(memory_space=pltpu.SMEM),
        out_shape=jax.ShapeDtypeStruct((1, 1), jnp.float32),
        scratch_shapes=[pltpu.SMEM((1,), jnp.float32)],
    )(src_t)


def _tc_corr_body(t, sub, idx_ref, g_ref, sum_ref, out_ref):
    slots_i = [idx_ref[pl.ds(j * sub, sub), :] for j in range(t)]
    tot = None
    for j in range(t):
        gj = jnp.clip(g_ref[pl.ds(j * sub, sub), :], _CLIP_LO, _CLIP_HI)
        fj = jnp.log(1.0 - gj) - jnp.log(gj)
        if j == 0:
            tot = fj
        else:
            ij = slots_i[j]
            dup = ij == slots_i[0]
            for k in range(1, j):
                dup = dup | (ij == slots_i[k])
            tot = tot + jnp.where(dup, 0.0, fj)
    out_ref[0, 0] = sum_ref[0, 0] + jnp.sum(tot)


def _tc_corr(t, idx_rows, g_rows, dense_sum):
    nrows, lanes = idx_rows.shape
    sub = nrows // t
    return pl.pallas_call(
        functools.partial(_tc_corr_body, t, sub),
        in_specs=[
            pl.BlockSpec((nrows, lanes), lambda: (0, 0)),
            pl.BlockSpec((nrows, lanes), lambda: (0, 0)),
            pl.BlockSpec(memory_space=pltpu.SMEM),
        ],
        out_specs=pl.BlockSpec(memory_space=pltpu.SMEM),
        out_shape=jax.ShapeDtypeStruct((1, 1), jnp.float32),
    )(idx_rows, g_rows, dense_sum)


def kernel(src, tgt_indices):
    b, v = src.shape
    t = tgt_indices.shape[1]
    idx32 = tgt_indices.astype(jnp.int32)

    cj = idx32.T.reshape(-1)
    pos = jnp.arange(b * t, dtype=jnp.int32)
    r1 = pos % b
    idx_flat = (cj // 8) * (8 * b) + (r1 // 128) * 1024 + (cj % 8) * 128 + (
        r1 % 128)

    src_flat = (
        src.T.reshape(v // 8, 8, b // _LANES, _LANES)
        .transpose(0, 2, 1, 3)
        .reshape(-1)
    )
    g_flat = _make_sc_gather(b * t)(src_flat, idx_flat)
    dense_sum = _tc_dense(src.T)

    total = _tc_corr(
        t,
        idx_flat.reshape(b * t // _LANES, _LANES),
        g_flat.reshape(b * t // _LANES, _LANES),
        dense_sum,
    )
    scale = jnp.float32(1.0 / (b * v))
    return total[0, 0] * scale


# speedup vs baseline: 1.0815x; 1.0815x over previous
import functools

import jax
import jax.numpy as jnp
from jax import lax
from jax.experimental import pallas as pl
from jax.experimental.pallas import tpu as pltpu
from jax.experimental.pallas import tpu_sc as plsc

_NC = 2
_NS = 16
_NW = _NC * _NS
_CH = 128

_CLIP_LO = 1e-8
_CLIP_HI = 1.0 - 1e-8
_LANES = 128


def _sc_gather_body(src_hbm, idx_hbm, out_hbm, idx_echo_hbm, idx_v, vals_v, sem):
    wid = lax.axis_index("s") * _NC + lax.axis_index("c")
    npw = idx_v.shape[0]
    span = pl.ds(wid * npw, npw)
    pltpu.sync_copy(idx_hbm.at[span], idx_v)
    pltpu.sync_copy(idx_v, idx_echo_hbm.at[span])
    nch = npw // _CH

    def fire(c, carry):
        sl = pl.ds(c * _CH, _CH)
        pltpu.async_copy(src_hbm.at[idx_v.at[sl]], vals_v.at[sl], sem)
        return carry

    def drain(c, carry):
        sl = pl.ds(c * _CH, _CH)
        pltpu.make_async_copy(src_hbm.at[idx_v.at[sl]], vals_v.at[sl], sem).wait()
        return carry

    lax.fori_loop(0, nch, fire, 0)
    lax.fori_loop(0, nch, drain, 0)
    pltpu.sync_copy(vals_v, out_hbm.at[span])


def _make_sc_gather(n_elems):
    npw = n_elems // _NW
    assert npw % _CH == 0
    assert (npw * _NW) == n_elems and npw % 8 == 0
    return functools.partial(
        pl.kernel,
        out_type=[
            jax.ShapeDtypeStruct((n_elems,), jnp.float32),
            jax.ShapeDtypeStruct((n_elems,), jnp.int32),
        ],
        mesh=plsc.VectorSubcoreMesh(core_axis_name="c", subcore_axis_name="s"),
        scratch_types=[
            pltpu.VMEM((npw,), jnp.int32),
            pltpu.VMEM((npw,), jnp.float32),
            pltpu.SemaphoreType.DMA,
        ],
    )(_sc_gather_body)


def _tc_dense_body(src_ref, sum_ref, acc_ref):
    i = pl.program_id(0)
    n_i = pl.num_programs(0)

    p = jnp.clip(src_ref[...], _CLIP_LO, _CLIP_HI)
    dense = jnp.sum(jnp.log(1.0 - p))

    @pl.when(i == 0)
    def _():
        acc_ref[0] = 0.0

    acc_ref[0] += -dense

    @pl.when(i == n_i - 1)
    def _():
        sum_ref[0, 0] = acc_ref[0]


def _tc_dense(src_t, block_rows=200):
    v, b = src_t.shape
    grid = (v // block_rows,)
    return pl.pallas_call(
        _tc_dense_body,
        grid=grid,
        in_specs=[pl.BlockSpec((block_rows, b), lambda i: (i, 0))],
        out_specs=pl.BlockSpec(memory_space=pltpu.SMEM),
        out_shape=jax.ShapeDtypeStruct((1, 1), jnp.float32),
        scratch_shapes=[pltpu.SMEM((1,), jnp.float32)],
    )(src_t)


def _tc_corr_body(t, sub, idx_ref, g_ref, sum_ref, out_ref):
    slots_i = [idx_ref[pl.ds(j * sub, sub), :] for j in range(t)]
    tot = None
    for j in range(t):
        gj = jnp.clip(g_ref[pl.ds(j * sub, sub), :], _CLIP_LO, _CLIP_HI)
        fj = jnp.log(1.0 - gj) - jnp.log(gj)
        if j == 0:
            tot = fj
        else:
            ij = slots_i[j]
            dup = ij == slots_i[0]
            for k in range(1, j):
                dup = dup | (ij == slots_i[k])
            tot = tot + jnp.where(dup, 0.0, fj)
    out_ref[0, 0] = sum_ref[0, 0] + jnp.sum(tot)


def _tc_corr(t, idx_rows, g_rows, dense_sum):
    nrows, lanes = idx_rows.shape
    sub = nrows // t
    return pl.pallas_call(
        functools.partial(_tc_corr_body, t, sub),
        in_specs=[
            pl.BlockSpec((nrows, lanes), lambda: (0, 0)),
            pl.BlockSpec((nrows, lanes), lambda: (0, 0)),
            pl.BlockSpec(memory_space=pltpu.SMEM),
        ],
        out_specs=pl.BlockSpec(memory_space=pltpu.SMEM),
        out_shape=jax.ShapeDtypeStruct((1, 1), jnp.float32),
    )(idx_rows, g_rows, dense_sum)


def kernel(src, tgt_indices):
    b, v = src.shape
    t = tgt_indices.shape[1]
    idx32 = tgt_indices.astype(jnp.int32)

    cj = idx32.T.reshape(-1)
    pos = jnp.arange(b * t, dtype=jnp.int32)
    r1 = pos % b
    idx_flat = (cj // 8) * (8 * b) + (r1 // 128) * 1024 + (cj % 8) * 128 + (
        r1 % 128)

    src_flat = (
        src.T.reshape(v // 8, 8, b // _LANES, _LANES)
        .transpose(0, 2, 1, 3)
        .reshape(-1)
    )
    g_flat, idx_echo = _make_sc_gather(b * t)(src_flat, idx_flat)
    dense_sum = _tc_dense(src.T)

    total = _tc_corr(
        t,
        idx_echo.reshape(b * t // _LANES, _LANES),
        g_flat.reshape(b * t // _LANES, _LANES),
        dense_sum,
    )
    scale = jnp.float32(1.0 / (b * v))
    return total[0, 0] * scale
